# baseline (speedup 1.0000x reference)
import functools

import jax
import jax.numpy as jnp
from jax.experimental import pallas as pl

_LANES = 128


def _decode_kernel(x_ref, o_ref, *, hw, img_w, img_h):
    def at(c):
        flat = c * hw
        return x_ref[flat // _LANES, flat % _LANES]

    cx, cy = at(0), at(1)
    dw, dh = at(2) * 0.5, at(3) * 0.5
    score = at(4)
    x1 = jnp.clip(cx - dw, 0.0, img_w)
    y1 = jnp.clip(cy - dh, 0.0, img_h)
    x2 = jnp.clip(cx + dw, 0.0, img_w)
    y2 = jnp.clip(cy + dh, 0.0, img_h)
    col = jax.lax.broadcasted_iota(jnp.int32, (1, 7), 1)
    out = jnp.zeros((1, 7), jnp.float32)
    for i, v in enumerate((x1, y1, x2, y2, score)):
        out = jnp.where(col == i + 1, v, out)
    o_ref[...] = out


def kernel(x):
    img_h, img_w = float(x.shape[2]), float(x.shape[3])
    hw = x.shape[2] * x.shape[3]
    xf = x.reshape(-1, _LANES)
    nrows = -(-(4 * hw + 1) // _LANES)
    nrows = -(-nrows // 8) * 8
    body = functools.partial(_decode_kernel, hw=hw, img_w=img_w, img_h=img_h)
    return pl.pallas_call(
        body,
        grid=(1,),
        in_specs=[pl.BlockSpec((nrows, _LANES), lambda i: (0, 0))],
        out_specs=pl.BlockSpec((1, 7), lambda i: (0, 0)),
        out_shape=jax.ShapeDtypeStruct((1, 7), jnp.float32),
    )(xf)


# speedup vs baseline: 3.1594x; 3.1594x over previous
import functools

import jax
import jax.numpy as jnp
from jax.experimental import pallas as pl


def _decode_kernel(cx_ref, cy_ref, w_ref, h_ref, s_ref, o_ref, *, img_w, img_h):
    cx = cx_ref[0, 0, 0, 0]
    cy = cy_ref[0, 0, 0, 0]
    dw = w_ref[0, 0, 0, 0] * 0.5
    dh = h_ref[0, 0, 0, 0] * 0.5
    score = s_ref[0, 0, 0, 0]
    x1 = jnp.clip(cx - dw, 0.0, img_w)
    y1 = jnp.clip(cy - dh, 0.0, img_h)
    x2 = jnp.clip(cx + dw, 0.0, img_w)
    y2 = jnp.clip(cy + dh, 0.0, img_h)
    col = jax.lax.broadcasted_iota(jnp.int32, (1, 7), 1)
    out = jnp.zeros((1, 7), jnp.float32)
    for i, v in enumerate((x1, y1, x2, y2, score)):
        out = jnp.where(col == i + 1, v, out)
    o_ref[...] = out


def _chan_spec(c):
    return pl.BlockSpec((1, 1, 8, 128), lambda i, _c=c: (0, _c, 0, 0))


def kernel(x):
    img_h, img_w = float(x.shape[2]), float(x.shape[3])
    body = functools.partial(_decode_kernel, img_w=img_w, img_h=img_h)
    return pl.pallas_call(
        body,
        grid=(1,),
        in_specs=[_chan_spec(c) for c in range(5)],
        out_specs=pl.BlockSpec((1, 7), lambda i: (0, 0)),
        out_shape=jax.ShapeDtypeStruct((1, 7), jnp.float32),
    )(x, x, x, x, x)


# speedup vs baseline: 3.1677x; 1.0026x over previous
import functools

import jax
import jax.numpy as jnp
from jax.experimental import pallas as pl
from jax.experimental.pallas import tpu as pltpu


def _decode_kernel(x_hbm, o_ref, vmem, sem, *, img_w, img_h):
    copies = [
        pltpu.make_async_copy(
            x_hbm.at[0, c, pl.ds(0, 8), pl.ds(0, 128)], vmem.at[c], sem
        )
        for c in range(5)
    ]
    for cp in copies:
        cp.start()
    for cp in copies:
        cp.wait()
    cx = vmem[0, 0, 0]
    cy = vmem[1, 0, 0]
    dw = vmem[2, 0, 0] * 0.5
    dh = vmem[3, 0, 0] * 0.5
    score = vmem[4, 0, 0]
    x1 = jnp.clip(cx - dw, 0.0, img_w)
    y1 = jnp.clip(cy - dh, 0.0, img_h)
    x2 = jnp.clip(cx + dw, 0.0, img_w)
    y2 = jnp.clip(cy + dh, 0.0, img_h)
    col = jax.lax.broadcasted_iota(jnp.int32, (1, 7), 1)
    out = jnp.zeros((1, 7), jnp.float32)
    for i, v in enumerate((x1, y1, x2, y2, score)):
        out = jnp.where(col == i + 1, v, out)
    o_ref[...] = out


def kernel(x):
    img_h, img_w = float(x.shape[2]), float(x.shape[3])
    body = functools.partial(_decode_kernel, img_w=img_w, img_h=img_h)
    return pl.pallas_call(
        body,
        grid=(1,),
        in_specs=[pl.BlockSpec(memory_space=pl.ANY)],
        out_specs=pl.BlockSpec((1, 7), lambda i: (0, 0)),
        out_shape=jax.ShapeDtypeStruct((1, 7), jnp.float32),
        scratch_shapes=[
            pltpu.VMEM((5, 8, 128), jnp.float32),
            pltpu.SemaphoreType.DMA,
        ],
    )(x)


# speedup vs baseline: 47.3853x; 14.9589x over previous
import functools

import jax
import jax.numpy as jnp
from jax.experimental import pallas as pl


def _decode_kernel(x_ref, o_ref, *, img_w, img_h):
    cx = x_ref[0, 0, 0, 0]
    cy = x_ref[0, 1, 0, 0]
    dw = x_ref[0, 2, 0, 0] * 0.5
    dh = x_ref[0, 3, 0, 0] * 0.5
    score = x_ref[0, 4, 0, 0]
    x1 = jnp.clip(cx - dw, 0.0, img_w)
    y1 = jnp.clip(cy - dh, 0.0, img_h)
    x2 = jnp.clip(cx + dw, 0.0, img_w)
    y2 = jnp.clip(cy + dh, 0.0, img_h)
    col = jax.lax.broadcasted_iota(jnp.int32, (1, 7), 1)
    out = jnp.zeros((1, 7), jnp.float32)
    for i, v in enumerate((x1, y1, x2, y2, score)):
        out = jnp.where(col == i + 1, v, out)
    o_ref[...] = out


def kernel(x):
    img_h, img_w = float(x.shape[2]), float(x.shape[3])
    tile = jax.lax.slice(x, (0, 0, 0, 0), (1, 5, 8, 128))
    body = functools.partial(_decode_kernel, img_w=img_w, img_h=img_h)
    return pl.pallas_call(
        body,
        grid=(1,),
        in_specs=[pl.BlockSpec((1, 5, 8, 128), lambda i: (0, 0, 0, 0))],
        out_specs=pl.BlockSpec((1, 7), lambda i: (0, 0)),
        out_shape=jax.ShapeDtypeStruct((1, 7), jnp.float32),
    )(tile)


# speedup vs baseline: 51.8243x; 1.0937x over previous
import functools

import jax
import jax.numpy as jnp
from jax.experimental import pallas as pl


def _decode_kernel(x_ref, o_ref, *, img_w, img_h):
    cx = x_ref[0, 0]
    cy = x_ref[1, 0]
    dw = x_ref[2, 0] * 0.5
    dh = x_ref[3, 0] * 0.5
    score = x_ref[4, 0]
    x1 = jnp.clip(cx - dw, 0.0, img_w)
    y1 = jnp.clip(cy - dh, 0.0, img_h)
    x2 = jnp.clip(cx + dw, 0.0, img_w)
    y2 = jnp.clip(cy + dh, 0.0, img_h)
    col = jax.lax.broadcasted_iota(jnp.int32, (1, 7), 1)
    out = jnp.zeros((1, 7), jnp.float32)
    for i, v in enumerate((x1, y1, x2, y2, score)):
        out = jnp.where(col == i + 1, v, out)
    o_ref[...] = out


def kernel(x):
    img_h, img_w = float(x.shape[2]), float(x.shape[3])
    tile = jax.lax.slice(x, (0, 0, 0, 0), (1, 5, 1, 128)).reshape(5, 128)
    body = functools.partial(_decode_kernel, img_w=img_w, img_h=img_h)
    return pl.pallas_call(
        body,
        grid=(1,),
        in_specs=[pl.BlockSpec((5, 128), lambda i: (0, 0))],
        out_specs=pl.BlockSpec((1, 7), lambda i: (0, 0)),
        out_shape=jax.ShapeDtypeStruct((1, 7), jnp.float32),
    )(tile)
